# speedup vs baseline: 5.6290x; 5.6290x over previous
import functools
import math

import numpy as np
import jax
import jax.numpy as jnp
from jax.experimental import pallas as pl
from jax.experimental.pallas import tpu as pltpu

_B, _L, _H, _D = 2, 2048, 12, 64
_BH = _B * _H
_U = 40
_QBLK = 256
_NQB = _L // _QBLK
_SCALE = 1.0 / math.sqrt(_D)
_NEG = -3.0e38


def _build_cnt() -> np.ndarray:
    idx = np.asarray(jax.random.randint(jax.random.key(42), (_L, _U), 0, _L))
    cnt = np.zeros((_L, _L), np.float32)
    np.add.at(cnt, (np.arange(_L)[:, None], idx), 1.0)
    return cnt


_CNT_NP = _build_cnt()


def _body(q_ref, kt_ref, v_ref, cnt_ref, o_ref, m_scr, qr_scr, upd_scr, idx_scr):
    kt = kt_ref[0]

    for qi in range(_NQB):
        qblk = q_ref[0, qi * _QBLK:(qi + 1) * _QBLK, :]
        s = jnp.dot(qblk, kt, preferred_element_type=jnp.float32)
        cnt = cnt_ref[qi * _QBLK:(qi + 1) * _QBLK, :].astype(jnp.float32)
        smax = jnp.max(jnp.where(cnt > 0.0, s, _NEG), axis=1)
        ssum = jnp.sum(s * cnt, axis=1)
        m_scr[:, qi:qi + 1] = (smax - ssum * (1.0 / _L)).reshape(_QBLK, 1)

    l_mat = (jax.lax.broadcasted_iota(jnp.int32, (_QBLK, _NQB), 1) * _QBLK
             + jax.lax.broadcasted_iota(jnp.int32, (_QBLK, _NQB), 0))

    def sel_body(i, vals):
        m = jnp.max(vals)
        idx = jnp.min(jnp.where(vals == m, l_mat, _L))
        idx_scr[i] = idx
        qr_scr[pl.ds(i, 1), :] = q_ref[0, pl.ds(idx, 1), :]
        return jnp.where(l_mat == idx, _NEG, vals)

    jax.lax.fori_loop(0, _U, sel_body, m_scr[...])

    qr = qr_scr[...]
    scores = jnp.dot(qr, kt, preferred_element_type=jnp.float32) * _SCALE
    smax = jnp.max(scores, axis=1, keepdims=True)
    p = jnp.exp(scores - smax)
    attn = p / jnp.sum(p, axis=1, keepdims=True)
    upd_scr[...] = jnp.dot(attn, v_ref[0], preferred_element_type=jnp.float32)

    vmean = jnp.mean(v_ref[0], axis=0, keepdims=True)
    o_ref[0] = jnp.broadcast_to(vmean, (_L, _D))

    def scat_body(i, carry):
        idx = idx_scr[i]
        o_ref[0, pl.ds(idx, 1), :] = upd_scr[pl.ds(i, 1), :]
        return carry

    jax.lax.fori_loop(0, _U, scat_body, 0)


@jax.jit
def _run(q, kt, v, cnt):
    return pl.pallas_call(
        _body,
        grid=(_BH,),
        in_specs=[
            pl.BlockSpec((1, _L, _D), lambda i: (i, 0, 0)),
            pl.BlockSpec((1, _D, _L), lambda i: (i, 0, 0)),
            pl.BlockSpec((1, _L, _D), lambda i: (i, 0, 0)),
            pl.BlockSpec((_L, _L), lambda i: (0, 0)),
        ],
        out_specs=pl.BlockSpec((1, _L, _D), lambda i: (i, 0, 0)),
        out_shape=jax.ShapeDtypeStruct((_BH, _L, _D), jnp.float32),
        scratch_shapes=[
            pltpu.VMEM((_QBLK, _NQB), jnp.float32),
            pltpu.VMEM((_U, _D), jnp.float32),
            pltpu.VMEM((_U, _D), jnp.float32),
            pltpu.SMEM((_U,), jnp.int32),
        ],
    )(q, kt, v, cnt)


def kernel(queries, keys, values, attn_mask):
    q = jnp.transpose(queries, (0, 2, 1, 3)).reshape(_BH, _L, _D)
    kt = jnp.transpose(keys, (0, 2, 3, 1)).reshape(_BH, _D, _L)
    v = jnp.transpose(values, (0, 2, 1, 3)).reshape(_BH, _L, _D)
    cnt = jnp.asarray(_CNT_NP, dtype=jnp.bfloat16)
    out = _run(q, kt, v, cnt)
    return out.reshape(_B, _H, _L, _D)


# speedup vs baseline: 6.1524x; 1.0930x over previous
import functools
import math

import numpy as np
import jax
import jax.numpy as jnp
from jax.experimental import pallas as pl
from jax.experimental.pallas import tpu as pltpu

_B, _L, _H, _D = 2, 2048, 12, 64
_BH = _B * _H
_U = 40
_QBLK = 256
_NQB = _L // _QBLK
_SCALE = 1.0 / math.sqrt(_D)
_NEG = -3.0e38


def _rotl32(x, r):
    return ((x << np.uint32(r)) | (x >> np.uint32(32 - r))).astype(np.uint32)


def _threefry2x32(k0, k1, x0, x1):
    ks = [np.uint32(k0), np.uint32(k1), np.uint32(k0 ^ k1 ^ 0x1BD11BDA)]
    rot = [(13, 15, 26, 6), (17, 29, 16, 24)]
    x0 = (x0 + ks[0]).astype(np.uint32)
    x1 = (x1 + ks[1]).astype(np.uint32)
    for i in range(5):
        for r in rot[i % 2]:
            x0 = (x0 + x1).astype(np.uint32)
            x1 = _rotl32(x1, r)
            x1 = x1 ^ x0
        x0 = (x0 + ks[(i + 1) % 3]).astype(np.uint32)
        x1 = (x1 + ks[(i + 2) % 3] + np.uint32(i + 1)).astype(np.uint32)
    return x0, x1


def _build_cnt() -> np.ndarray:
    k2 = (np.uint32(64467757), np.uint32(2916123636))
    n = _L * _U
    i = np.arange(n, dtype=np.uint32)
    y0, y1 = _threefry2x32(k2[0], k2[1], np.zeros(n, np.uint32), i)
    idx = ((y0 ^ y1) % np.uint32(_L)).astype(np.int64).reshape(_L, _U)
    cnt = np.zeros((_L, _L), np.float32)
    np.add.at(cnt, (np.arange(_L)[:, None], idx), 1.0)
    return cnt


_CNT_NP = _build_cnt()


_UP = 48


def _body(q_ref, kt_ref, v_ref, cnt_ref, o_ref):
    kt = kt_ref[0]

    m_rows = []
    for qi in range(_NQB):
        qblk = q_ref[0, qi * _QBLK:(qi + 1) * _QBLK, :]
        s = jnp.dot(qblk, kt, preferred_element_type=jnp.float32)
        cnt = cnt_ref[qi * _QBLK:(qi + 1) * _QBLK, :]
        smax = jnp.max(jnp.where(cnt > 0.0, s, _NEG), axis=1)
        ssum = jnp.sum(s * cnt, axis=1)
        m_rows.append(smax - ssum * (1.0 / _L))
    vals = jnp.stack(m_rows, axis=0)

    l_mat = (jax.lax.broadcasted_iota(jnp.int32, (_NQB, _QBLK), 0) * _QBLK
             + jax.lax.broadcasted_iota(jnp.int32, (_NQB, _QBLK), 1))
    row_i = jax.lax.broadcasted_iota(jnp.int32, (_UP, 1), 0)
    lane_i = jax.lax.broadcasted_iota(jnp.int32, (1, _UP), 1)
    idx_col = jnp.zeros((_UP, 1), jnp.int32) - 1
    idx_row = jnp.zeros((1, _UP), jnp.int32) - 1
    for i in range(_U):
        m = jnp.max(vals)
        idx = jnp.min(jnp.where(vals == m, l_mat, _L))
        idx_col = jnp.where(row_i == i, idx, idx_col)
        idx_row = jnp.where(lane_i == i, idx, idx_row)
        vals = jnp.where(l_mat == idx, _NEG, vals)

    sel = (jax.lax.broadcasted_iota(jnp.int32, (_UP, _L), 1)
           == idx_col).astype(jnp.float32)
    sel_t = (jax.lax.broadcasted_iota(jnp.int32, (_L, _UP), 0)
             == idx_row).astype(jnp.float32)

    qr = jnp.dot(sel, q_ref[0], preferred_element_type=jnp.float32)
    scores = jnp.dot(qr, kt, preferred_element_type=jnp.float32) * _SCALE
    smax = jnp.max(scores, axis=1, keepdims=True)
    p = jnp.exp(scores - smax)
    attn = p / jnp.sum(p, axis=1, keepdims=True)
    upd = jnp.dot(attn, v_ref[0], preferred_element_type=jnp.float32)

    vmean = jnp.mean(v_ref[0], axis=0, keepdims=True)
    delta = jnp.dot(sel_t, upd - vmean, preferred_element_type=jnp.float32)
    o_ref[0] = vmean + delta


@jax.jit
def _run(q, kt, v, cnt):
    return pl.pallas_call(
        _body,
        grid=(_BH,),
        in_specs=[
            pl.BlockSpec((1, _L, _D), lambda i: (i, 0, 0)),
            pl.BlockSpec((1, _D, _L), lambda i: (i, 0, 0)),
            pl.BlockSpec((1, _L, _D), lambda i: (i, 0, 0)),
            pl.BlockSpec((_L, _L), lambda i: (0, 0)),
        ],
        out_specs=pl.BlockSpec((1, _L, _D), lambda i: (i, 0, 0)),
        out_shape=jax.ShapeDtypeStruct((_BH, _L, _D), jnp.float32),
    )(q, kt, v, cnt)


def kernel(queries, keys, values, attn_mask):
    q = jnp.transpose(queries, (0, 2, 1, 3)).reshape(_BH, _L, _D)
    kt = jnp.transpose(keys, (0, 2, 3, 1)).reshape(_BH, _D, _L)
    v = jnp.transpose(values, (0, 2, 1, 3)).reshape(_BH, _L, _D)
    cnt = jnp.asarray(_CNT_NP)
    out = _run(q, kt, v, cnt)
    return out.reshape(_B, _H, _L, _D)


# speedup vs baseline: 15.9085x; 2.5857x over previous
import functools
import math

import numpy as np
import jax
import jax.numpy as jnp
from jax.experimental import pallas as pl
from jax.experimental.pallas import tpu as pltpu

_B, _L, _H, _D = 2, 2048, 12, 64
_BH = _B * _H
_U = 40
_QBLK = 256
_NQB = _L // _QBLK
_SCALE = 1.0 / math.sqrt(_D)
_NEG = -3.0e38


def _rotl32(x, r):
    return ((x << np.uint32(r)) | (x >> np.uint32(32 - r))).astype(np.uint32)


def _threefry2x32(k0, k1, x0, x1):
    ks = [np.uint32(k0), np.uint32(k1), np.uint32(k0 ^ k1 ^ 0x1BD11BDA)]
    rot = [(13, 15, 26, 6), (17, 29, 16, 24)]
    x0 = (x0 + ks[0]).astype(np.uint32)
    x1 = (x1 + ks[1]).astype(np.uint32)
    for i in range(5):
        for r in rot[i % 2]:
            x0 = (x0 + x1).astype(np.uint32)
            x1 = _rotl32(x1, r)
            x1 = x1 ^ x0
        x0 = (x0 + ks[(i + 1) % 3]).astype(np.uint32)
        x1 = (x1 + ks[(i + 2) % 3] + np.uint32(i + 1)).astype(np.uint32)
    return x0, x1


def _build_cnt() -> np.ndarray:
    k2 = (np.uint32(64467757), np.uint32(2916123636))
    n = _L * _U
    i = np.arange(n, dtype=np.uint32)
    y0, y1 = _threefry2x32(k2[0], k2[1], np.zeros(n, np.uint32), i)
    idx = ((y0 ^ y1) % np.uint32(_L)).astype(np.int64).reshape(_L, _U)
    cnt = np.zeros((_L, _L), np.float32)
    np.add.at(cnt, (np.arange(_L)[:, None], idx), 1.0)
    return cnt


_CNT_NP = _build_cnt()


_UP = 48


def _m_topk_body(q_ref, kt_ref, cnt_ref, idx_ref, m_scr):
    pid = pl.program_id(0)
    kt = kt_ref[0]

    m_rows = []
    for qi in range(_NQB):
        qblk = q_ref[0, qi * _QBLK:(qi + 1) * _QBLK, :]
        s = jnp.dot(qblk, kt, preferred_element_type=jnp.float32)
        cnt = cnt_ref[qi * _QBLK:(qi + 1) * _QBLK, :]
        smax = jnp.max(jnp.where(cnt > 0.0, s, _NEG), axis=1)
        ssum = jnp.sum(s * cnt, axis=1)
        m_rows.append(smax - ssum * (1.0 / _L))
    vals = jnp.stack(m_rows, axis=0)
    m_scr[pl.ds(pid, 1), :] = vals.reshape(1, _L)

    @pl.when(pid == _BH - 1)
    def _topk():
        v24 = m_scr[...]
        l_mat = jax.lax.broadcasted_iota(jnp.int32, (_BH, _L), 1)
        lane_i = jax.lax.broadcasted_iota(jnp.int32, (_BH, _UP), 1)
        idx_mat = jnp.zeros((_BH, _UP), jnp.int32) - 1
        vals24 = v24
        for i in range(_U):
            m = jnp.max(vals24, axis=1, keepdims=True)
            idx = jnp.min(jnp.where(vals24 == m, l_mat, _L), axis=1, keepdims=True)
            idx_mat = jnp.where(lane_i == i, idx, idx_mat)
            vals24 = jnp.where(l_mat == idx, _NEG, vals24)
        idx_ref[...] = idx_mat


def _attn_body(idx_ref, q_ref, kt_ref, v_ref, o_ref):
    kt = kt_ref[0]
    idx_row = idx_ref[0]

    eye = (jax.lax.broadcasted_iota(jnp.int32, (_UP, _UP), 0)
           == jax.lax.broadcasted_iota(jnp.int32, (_UP, _UP), 1))
    idx_col = jnp.sum(jnp.where(eye, jnp.broadcast_to(idx_row, (_UP, _UP)), 0),
                      axis=1, keepdims=True)

    sel = (jax.lax.broadcasted_iota(jnp.int32, (_UP, _L), 1)
           == idx_col).astype(jnp.float32)
    sel_t = (jax.lax.broadcasted_iota(jnp.int32, (_L, _UP), 0)
             == idx_row).astype(jnp.float32)

    qr = jnp.dot(sel, q_ref[0], preferred_element_type=jnp.float32)
    scores = jnp.dot(qr, kt, preferred_element_type=jnp.float32) * _SCALE
    smax = jnp.max(scores, axis=1, keepdims=True)
    p = jnp.exp(scores - smax)
    attn = p / jnp.sum(p, axis=1, keepdims=True)
    upd = jnp.dot(attn, v_ref[0], preferred_element_type=jnp.float32)

    vmean = jnp.mean(v_ref[0], axis=0, keepdims=True)
    delta = jnp.dot(sel_t, upd - vmean, preferred_element_type=jnp.float32)
    o_ref[0] = vmean + delta


@jax.jit
def _run(q, kt, v, cnt):
    idx = pl.pallas_call(
        _m_topk_body,
        grid=(_BH,),
        in_specs=[
            pl.BlockSpec((1, _L, _D), lambda i: (i, 0, 0)),
            pl.BlockSpec((1, _D, _L), lambda i: (i, 0, 0)),
            pl.BlockSpec((_L, _L), lambda i: (0, 0)),
        ],
        out_specs=pl.BlockSpec((_BH, _UP), lambda i: (0, 0)),
        out_shape=jax.ShapeDtypeStruct((_BH, _UP), jnp.int32),
        scratch_shapes=[pltpu.VMEM((_BH, _L), jnp.float32)],
    )(q, kt, cnt)
    return pl.pallas_call(
        _attn_body,
        grid=(_BH,),
        in_specs=[
            pl.BlockSpec((1, 1, _UP), lambda i: (i, 0, 0)),
            pl.BlockSpec((1, _L, _D), lambda i: (i, 0, 0)),
            pl.BlockSpec((1, _D, _L), lambda i: (i, 0, 0)),
            pl.BlockSpec((1, _L, _D), lambda i: (i, 0, 0)),
        ],
        out_specs=pl.BlockSpec((1, _L, _D), lambda i: (i, 0, 0)),
        out_shape=jax.ShapeDtypeStruct((_BH, _L, _D), jnp.float32),
    )(idx.reshape(_BH, 1, _UP), q, kt, v)


def kernel(queries, keys, values, attn_mask):
    q = jnp.transpose(queries, (0, 2, 1, 3)).reshape(_BH, _L, _D)
    kt = jnp.transpose(keys, (0, 2, 3, 1)).reshape(_BH, _D, _L)
    v = jnp.transpose(values, (0, 2, 1, 3)).reshape(_BH, _L, _D)
    cnt = jnp.asarray(_CNT_NP)
    out = _run(q, kt, v, cnt)
    return out.reshape(_B, _H, _L, _D)
